# baseline (speedup 1.0000x reference)
import jax
import jax.numpy as jnp
from jax.experimental import pallas as pl
from jax.experimental.pallas import tpu as pltpu

E = 8
FFN = 4096
H = 1024
T = 512
BF = 512
NCORES = 2
NBF = FFN // BF // NCORES


def _glu_body(eidx_ref, x_ref, w1_ref, v1_ref, w2_ref, o_ref):
    f = pl.program_id(1)
    x = x_ref[...].astype(jnp.bfloat16)
    h1 = jax.lax.dot_general(
        x, w1_ref[...].astype(jnp.bfloat16), (((1,), (1,)), ((), ())),
        preferred_element_type=jnp.float32)
    h2 = jax.lax.dot_general(
        x, v1_ref[...].astype(jnp.bfloat16), (((1,), (1,)), ((), ())),
        preferred_element_type=jnp.float32)
    g = h1 * jax.lax.logistic(h1) * h2
    contrib = jnp.dot(g.astype(jnp.bfloat16), w2_ref[...].astype(jnp.bfloat16),
                      preferred_element_type=jnp.float32)

    @pl.when(f == 0)
    def _():
        o_ref[...] = contrib[None]

    @pl.when(f != 0)
    def _():
        o_ref[...] = o_ref[...] + contrib[None]


def _sum_body(p_ref, o_ref):
    o_ref[...] = p_ref[0] + p_ref[1]


def kernel(x, expert_idx, w1, v1, w2):
    eidx = jnp.asarray(expert_idx, dtype=jnp.int32).reshape((1,))

    def _w_map(c, f, e):
        return (e[0] * (NCORES * NBF) + c * NBF + f, 0)

    grid_spec = pltpu.PrefetchScalarGridSpec(
        num_scalar_prefetch=1,
        grid=(NCORES, NBF),
        in_specs=[
            pl.BlockSpec((T, H), lambda c, f, e: (0, 0)),
            pl.BlockSpec((BF, H), _w_map),
            pl.BlockSpec((BF, H), _w_map),
            pl.BlockSpec((BF, H), _w_map),
        ],
        out_specs=pl.BlockSpec((1, T, H), lambda c, f, e: (c, 0, 0)),
    )

    partials = pl.pallas_call(
        _glu_body,
        grid_spec=grid_spec,
        out_shape=jax.ShapeDtypeStruct((NCORES, T, H), jnp.float32),
        compiler_params=pltpu.CompilerParams(
            dimension_semantics=("parallel", "arbitrary")),
    )(eidx, x, w1, v1, w2)

    return pl.pallas_call(
        _sum_body,
        out_shape=jax.ShapeDtypeStruct((T, H), jnp.float32),
    )(partials)


# speedup vs baseline: 1.0014x; 1.0014x over previous
import jax
import jax.numpy as jnp
from jax.experimental import pallas as pl
from jax.experimental.pallas import tpu as pltpu

E = 8
FFN = 4096
H = 1024
T = 512
BF = 256
NBF = FFN // BF


def _glu_body(eidx_ref, x_ref, w1_ref, v1_ref, w2_ref, o_ref):
    f = pl.program_id(0)
    x = x_ref[...].astype(jnp.bfloat16)
    h1 = jax.lax.dot_general(
        x, w1_ref[...].astype(jnp.bfloat16), (((1,), (1,)), ((), ())),
        preferred_element_type=jnp.float32)
    h2 = jax.lax.dot_general(
        x, v1_ref[...].astype(jnp.bfloat16), (((1,), (1,)), ((), ())),
        preferred_element_type=jnp.float32)
    g = h1 * jax.lax.logistic(h1) * h2
    contrib = jnp.dot(g.astype(jnp.bfloat16), w2_ref[...].astype(jnp.bfloat16),
                      preferred_element_type=jnp.float32)

    @pl.when(f == 0)
    def _():
        o_ref[...] = contrib

    @pl.when(f != 0)
    def _():
        o_ref[...] = o_ref[...] + contrib


def kernel(x, expert_idx, w1, v1, w2):
    eidx = jnp.asarray(expert_idx, dtype=jnp.int32).reshape((1,))

    def _w_map(f, e):
        return (e[0] * NBF + f, 0)

    grid_spec = pltpu.PrefetchScalarGridSpec(
        num_scalar_prefetch=1,
        grid=(NBF,),
        in_specs=[
            pl.BlockSpec((T, H), lambda f, e: (0, 0)),
            pl.BlockSpec((BF, H), _w_map),
            pl.BlockSpec((BF, H), _w_map),
            pl.BlockSpec((BF, H), _w_map),
        ],
        out_specs=pl.BlockSpec((T, H), lambda f, e: (0, 0)),
    )

    return pl.pallas_call(
        _glu_body,
        grid_spec=grid_spec,
        out_shape=jax.ShapeDtypeStruct((T, H), jnp.float32),
        compiler_params=pltpu.CompilerParams(
            dimension_semantics=("arbitrary",)),
    )(eidx, x, w1, v1, w2)


# speedup vs baseline: 1.1952x; 1.1935x over previous
import jax
import jax.numpy as jnp
from jax.experimental import pallas as pl
from jax.experimental.pallas import tpu as pltpu

E = 8
FFN = 4096
H = 1024
T = 512
BF = 1024
NBF = FFN // BF


def _glu_body(eidx_ref, x_ref, w1_ref, v1_ref, w2_ref, o_ref):
    f = pl.program_id(0)
    x = x_ref[...].astype(jnp.bfloat16)
    h1 = jax.lax.dot_general(
        x, w1_ref[...].astype(jnp.bfloat16), (((1,), (1,)), ((), ())),
        preferred_element_type=jnp.float32)
    h2 = jax.lax.dot_general(
        x, v1_ref[...].astype(jnp.bfloat16), (((1,), (1,)), ((), ())),
        preferred_element_type=jnp.float32)
    g = h1 * jax.lax.logistic(h1) * h2
    contrib = jnp.dot(g.astype(jnp.bfloat16), w2_ref[...].astype(jnp.bfloat16),
                      preferred_element_type=jnp.float32)

    @pl.when(f == 0)
    def _():
        o_ref[...] = contrib

    @pl.when(f != 0)
    def _():
        o_ref[...] = o_ref[...] + contrib


def kernel(x, expert_idx, w1, v1, w2):
    eidx = jnp.asarray(expert_idx, dtype=jnp.int32).reshape((1,))

    def _w_map(f, e):
        return (e[0] * NBF + f, 0)

    grid_spec = pltpu.PrefetchScalarGridSpec(
        num_scalar_prefetch=1,
        grid=(NBF,),
        in_specs=[
            pl.BlockSpec((T, H), lambda f, e: (0, 0)),
            pl.BlockSpec((BF, H), _w_map),
            pl.BlockSpec((BF, H), _w_map),
            pl.BlockSpec((BF, H), _w_map),
        ],
        out_specs=pl.BlockSpec((T, H), lambda f, e: (0, 0)),
    )

    return pl.pallas_call(
        _glu_body,
        grid_spec=grid_spec,
        out_shape=jax.ShapeDtypeStruct((T, H), jnp.float32),
        compiler_params=pltpu.CompilerParams(
            dimension_semantics=("arbitrary",)),
    )(eidx, x, w1, v1, w2)


# speedup vs baseline: 1.2087x; 1.0113x over previous
import jax
import jax.numpy as jnp
from jax.experimental import pallas as pl
from jax.experimental.pallas import tpu as pltpu

E = 8
FFN = 4096
H = 1024
T = 512
BF = 1024
NBF = FFN // BF


def _glu_body(eidx_ref, x_ref, w1_ref, v1_ref, w2_ref, o_ref):
    f = pl.program_id(0)
    x = x_ref[...]
    h1 = jax.lax.dot_general(
        x, w1_ref[...], (((1,), (1,)), ((), ())),
        preferred_element_type=jnp.float32)
    h2 = jax.lax.dot_general(
        x, v1_ref[...], (((1,), (1,)), ((), ())),
        preferred_element_type=jnp.float32)
    g = h1 * jax.lax.logistic(h1) * h2
    contrib = jnp.dot(g, w2_ref[...], preferred_element_type=jnp.float32)

    @pl.when(f == 0)
    def _():
        o_ref[...] = contrib

    @pl.when(f != 0)
    def _():
        o_ref[...] = o_ref[...] + contrib


def kernel(x, expert_idx, w1, v1, w2):
    eidx = jnp.asarray(expert_idx, dtype=jnp.int32).reshape((1,))

    def _w_map(f, e):
        return (e[0] * NBF + f, 0)

    grid_spec = pltpu.PrefetchScalarGridSpec(
        num_scalar_prefetch=1,
        grid=(NBF,),
        in_specs=[
            pl.BlockSpec((T, H), lambda f, e: (0, 0)),
            pl.BlockSpec((BF, H), _w_map),
            pl.BlockSpec((BF, H), _w_map),
            pl.BlockSpec((BF, H), _w_map),
        ],
        out_specs=pl.BlockSpec((T, H), lambda f, e: (0, 0)),
    )

    return pl.pallas_call(
        _glu_body,
        grid_spec=grid_spec,
        out_shape=jax.ShapeDtypeStruct((T, H), jnp.float32),
        compiler_params=pltpu.CompilerParams(
            dimension_semantics=("arbitrary",)),
    )(eidx, x, w1, v1, w2)
